# baseline (speedup 1.0000x reference)
import functools

import jax
import jax.numpy as jnp
from jax import lax
from jax.experimental import pallas as pl
from jax.experimental.pallas import tpu as pltpu
from jax.experimental.pallas import tpu_sc as plsc

N = 10000
E = 320000
F_IN = 128
HID = 64
MU = 0.1
EPS = 1e-6
K_ITERS = 4

NW = 32
NP = 10240
EPT = NP
KC = 80
C = 128
STRIPE = NP // 16

_MESH = plsc.VectorSubcoreMesh(core_axis_name="c", subcore_axis_name="s")


def _nrsqrt(v):
    i = lax.bitcast_convert_type(v, jnp.int32)
    i = jnp.int32(0x5F3759DF) - lax.shift_right_logical(i, 1)
    y = lax.bitcast_convert_type(i, jnp.float32)
    for _ in range(2):
        y = y * (1.5 - 0.5 * v * y * y)
    return y


def _lin1_body(x_ref, w_ref, b_ref, o_ref):
    o_ref[...] = jax.nn.relu(
        jnp.dot(x_ref[...], w_ref[...], preferred_element_type=jnp.float32)
        + b_ref[...])


def _lin1(x_pad, W1, b1):
    blk = NP // 8
    return pl.pallas_call(
        _lin1_body,
        grid=(8,),
        in_specs=[
            pl.BlockSpec((blk, F_IN), lambda i: (i, 0)),
            pl.BlockSpec((F_IN, HID), lambda i: (0, 0)),
            pl.BlockSpec((1, HID), lambda i: (0, 0)),
        ],
        out_specs=pl.BlockSpec((blk, HID), lambda i: (i, 0)),
        out_shape=jax.ShapeDtypeStruct((NP, HID), jnp.float32),
    )(x_pad, W1, b1.reshape(1, HID))


def _out_body(u_ref, di_ref, w_ref, b_ref, o_ref):
    f = u_ref[...].astype(jnp.float32) / di_ref[...]
    y = jnp.dot(f, w_ref[...], preferred_element_type=jnp.float32) + b_ref[...]
    m = jnp.max(y, axis=1, keepdims=True)
    ex = jnp.exp(y - m)
    lse = jnp.log(jnp.sum(ex, axis=1, keepdims=True))
    o_ref[...] = y - m - lse


def _lin2(u, dinv_col, W2, b2):
    blk = 1000
    return pl.pallas_call(
        _out_body,
        grid=(10,),
        in_specs=[
            pl.BlockSpec((blk, HID), lambda i: (i, 0), ),
            pl.BlockSpec((blk, 1), lambda i: (i, 0)),
            pl.BlockSpec((HID, HID), lambda i: (0, 0)),
            pl.BlockSpec((1, HID), lambda i: (0, 0)),
        ],
        out_specs=pl.BlockSpec((blk, HID), lambda i: (i, 0)),
        out_shape=jax.ShapeDtypeStruct((N, HID), jnp.float32),
    )(u, dinv_col, W2, b2.reshape(1, HID))


def _deg_body(dst_hbm, deg_out, dst_v, ones_v, zbuf, sem, deg_sp):
    c = lax.axis_index("c")
    s = lax.axis_index("s")
    w = c * 16 + s
    pltpu.sync_copy(dst_hbm.at[w], dst_v)
    for f in range(8):
        ones_v[pl.ds(f * 16, 16)] = jnp.ones((16,), jnp.float32)
    for f in range(STRIPE // 16):
        zbuf[pl.ds(f * 16, 16)] = jnp.zeros((16,), jnp.float32)
    stripe = s * STRIPE
    pltpu.sync_copy(zbuf, deg_sp.at[pl.ds(stripe, STRIPE)])
    plsc.subcore_barrier()

    def chunk(k8, carry):
        descs = [
            pltpu.async_copy(
                ones_v, deg_sp.at[dst_v.at[k8 * 8 + j]], sem, add=True)
            for j in range(8)
        ]
        for d in descs:
            d.wait()
        return carry

    lax.fori_loop(0, KC // 8, chunk, 0)
    plsc.subcore_barrier()
    pltpu.sync_copy(deg_sp.at[pl.ds(stripe, STRIPE)], zbuf)
    pltpu.sync_copy(zbuf, deg_out.at[pl.ds(c * NP + stripe, STRIPE)])


def _deg(dst_blk):
    k = pl.kernel(
        _deg_body,
        out_type=jax.ShapeDtypeStruct((2 * NP,), jnp.float32),
        mesh=_MESH,
        compiler_params=pltpu.CompilerParams(needs_layout_passes=False, use_tc_tiling_on_sc=False),
        scratch_types=[
            pltpu.VMEM((KC, C), jnp.int32),
            pltpu.VMEM((C,), jnp.float32),
            pltpu.VMEM((STRIPE,), jnp.float32),
            pltpu.SemaphoreType.DMA,
            pltpu.VMEM_SHARED((NP,), jnp.float32),
        ],
    )
    return k(dst_blk)


def _prep_body(deg_hbm, h_hbm, dinv_out, u0_out, d0_v, d1_v, h_v, di_v, u0_v):
    c = lax.axis_index("c")
    s = lax.axis_index("s")
    w = c * 16 + s
    base = w * (NP // NW)
    rows = NP // NW
    pltpu.sync_copy(deg_hbm.at[pl.ds(base, rows)], d0_v)
    pltpu.sync_copy(deg_hbm.at[pl.ds(NP + base, rows)], d1_v)
    pltpu.sync_copy(h_hbm.at[pl.ds(base, rows)], h_v)

    def grp(g, carry):
        d = d0_v[pl.ds(g * 16, 16)] + d1_v[pl.ds(g * 16, 16)]
        d = jnp.maximum(d, 1.0)
        dv = _nrsqrt(d)
        di_v[pl.ds(g * 16, 16)] = dv
        for el in range(16):
            i = g * 16 + el
            de = dv[el]
            for f in range(HID // 16):
                u0_v[i, pl.ds(f * 16, 16)] = de * h_v[i, pl.ds(f * 16, 16)]
        return carry

    lax.fori_loop(0, rows // 16, grp, 0)
    pltpu.sync_copy(di_v, dinv_out.at[pl.ds(base, rows)])
    pltpu.sync_copy(u0_v, u0_out.at[pl.ds(base, rows)])


def _prep(deg_pair, h):
    rows = NP // NW
    k = pl.kernel(
        _prep_body,
        out_type=(jax.ShapeDtypeStruct((NP,), jnp.float32),
                  jax.ShapeDtypeStruct((NP, HID), jnp.float32)),
        mesh=_MESH,
        compiler_params=pltpu.CompilerParams(needs_layout_passes=False, use_tc_tiling_on_sc=False),
        scratch_types=[
            pltpu.VMEM((rows,), jnp.float32),
            pltpu.VMEM((rows,), jnp.float32),
            pltpu.VMEM((rows, HID), jnp.float32),
            pltpu.VMEM((rows,), jnp.float32),
            pltpu.VMEM((rows, HID), jnp.float32),
        ],
    )
    return k(deg_pair, h)


def _edge_body(u_hbm, dinv_hbm, src_hbm, dst_hbm, t_out, r_out,
               src_v, dst_v, dinv_vm, us_v0, us_v1, ud_v0, ud_v1,
               msg_v0, msg_v1, pbuf, rbuf0, rbuf1, zbuf,
               gs0, gs1, ss0, ss1, t_sp, r_sp):
    us_v = (us_v0, us_v1)
    ud_v = (ud_v0, ud_v1)
    msg_v = (msg_v0, msg_v1)
    rbuf = (rbuf0, rbuf1)
    gs = (gs0, gs1)
    ss = (ss0, ss1)
    c = lax.axis_index("c")
    s = lax.axis_index("s")
    w = c * 16 + s
    pltpu.sync_copy(dinv_hbm, dinv_vm)
    pltpu.sync_copy(src_hbm.at[w], src_v)
    pltpu.sync_copy(dst_hbm.at[w], dst_v)

    def zmsg(e, carry):
        for f in range(HID // 16):
            msg_v0[e, pl.ds(f * 16, 16)] = jnp.zeros((16,), jnp.float32)
        return carry

    lax.fori_loop(0, C, zmsg, 0)
    for f in range(STRIPE // 16):
        zbuf[pl.ds(f * 16, 16)] = jnp.zeros((16,), jnp.float32)
    stripe = s * STRIPE
    for j in range(STRIPE // C):
        pltpu.sync_copy(msg_v0, t_sp.at[pl.ds(stripe + j * C, C)])
    pltpu.sync_copy(zbuf, r_sp.at[pl.ds(stripe, STRIPE)])
    plsc.subcore_barrier()

    iot = jnp.arange(16, dtype=jnp.int32)

    def compute_chunk(k, usb, udb, msgb, rbb):
        @plsc.parallel_loop(0, C // 16, unroll=4)
        def grp(g):
            eb = g * 16
            pb = g * 256
            for el in range(16):
                e = eb + el
                acc = jnp.zeros((16,), jnp.float32)
                for f in range(HID // 16):
                    d = usb[e, pl.ds(f * 16, 16)] - udb[e, pl.ds(f * 16, 16)]
                    acc = acc + d * d
                pbuf[pl.ds(pb + el * 16, 16)] = acc
            iot16 = iot * 16 + pb
            cols = [plsc.load_gather(pbuf, [iot16 + j]) for j in range(16)]
            while len(cols) > 1:
                cols = [a + b for a, b in zip(cols[::2], cols[1::2])]
            gc = jnp.maximum(cols[0], EPS)
            m = _nrsqrt(gc * _nrsqrt(gc))
            sv = src_v[k, pl.ds(eb, 16)]
            dsv = plsc.load_gather(dinv_vm, [sv])
            rbb[pl.ds(eb, 16)] = m * dsv
            for el in range(16):
                e = eb + el
                me = m[el]
                for f in range(HID // 16):
                    msgb[e, pl.ds(f * 16, 16)] = me * usb[e, pl.ds(f * 16, 16)]

    pltpu.async_copy(u_hbm.at[src_v.at[0]], us_v[0], gs[0])
    pltpu.async_copy(u_hbm.at[dst_v.at[0]], ud_v[0], gs[0])

    def chunk2(k2, carry):
        for b in range(2):
            k = k2 * 2 + b
            pltpu.make_async_copy(u_hbm.at[src_v.at[k]], us_v[b], gs[b]).wait()
            pltpu.make_async_copy(u_hbm.at[dst_v.at[k]], ud_v[b], gs[b]).wait()

            @pl.when(k + 1 < KC)
            def _fire_gather():
                pltpu.async_copy(u_hbm.at[src_v.at[k + 1]], us_v[1 - b], gs[1 - b])
                pltpu.async_copy(u_hbm.at[dst_v.at[k + 1]], ud_v[1 - b], gs[1 - b])

            @pl.when(k2 > 0)
            def _drain_scatter():
                pltpu.make_async_copy(
                    msg_v[b], t_sp.at[dst_v.at[k - 2]], ss[b]).wait()
                pltpu.make_async_copy(
                    rbuf[b], r_sp.at[dst_v.at[k - 2]], ss[b]).wait()

            compute_chunk(k, us_v[b], ud_v[b], msg_v[b], rbuf[b])
            pltpu.async_copy(msg_v[b], t_sp.at[dst_v.at[k]], ss[b], add=True)
            pltpu.async_copy(rbuf[b], r_sp.at[dst_v.at[k]], ss[b], add=True)
        return carry

    lax.fori_loop(0, KC // 2, chunk2, 0)
    for b in range(2):
        pltpu.make_async_copy(
            msg_v[b], t_sp.at[dst_v.at[KC - 2 + b]], ss[b]).wait()
        pltpu.make_async_copy(
            rbuf[b], r_sp.at[dst_v.at[KC - 2 + b]], ss[b]).wait()
    plsc.subcore_barrier()
    pltpu.sync_copy(t_sp.at[pl.ds(stripe, STRIPE)],
                    t_out.at[c, pl.ds(stripe, STRIPE)])
    pltpu.sync_copy(r_sp.at[pl.ds(stripe, STRIPE)],
                    r_out.at[pl.ds(c * NP + stripe, STRIPE)])


def _edge(u, dinv, src_blk, dst_blk):
    k = pl.kernel(
        _edge_body,
        out_type=(jax.ShapeDtypeStruct((2, NP, HID), jnp.float32),
                  jax.ShapeDtypeStruct((2 * NP,), jnp.float32)),
        mesh=_MESH,
        compiler_params=pltpu.CompilerParams(needs_layout_passes=False, use_tc_tiling_on_sc=False),
        scratch_types=[
            pltpu.VMEM((KC, C), jnp.int32),
            pltpu.VMEM((KC, C), jnp.int32),
            pltpu.VMEM((NP,), jnp.float32),
            pltpu.VMEM((C, HID), jnp.float32),
            pltpu.VMEM((C, HID), jnp.float32),
            pltpu.VMEM((C, HID), jnp.float32),
            pltpu.VMEM((C, HID), jnp.float32),
            pltpu.VMEM((C, HID), jnp.float32),
            pltpu.VMEM((C, HID), jnp.float32),
            pltpu.VMEM((2048,), jnp.float32),
            pltpu.VMEM((C,), jnp.float32),
            pltpu.VMEM((C,), jnp.float32),
            pltpu.VMEM((STRIPE,), jnp.float32),
            pltpu.SemaphoreType.DMA,
            pltpu.SemaphoreType.DMA,
            pltpu.SemaphoreType.DMA,
            pltpu.SemaphoreType.DMA,
            pltpu.VMEM_SHARED((NP, HID), jnp.float32),
            pltpu.VMEM_SHARED((NP,), jnp.float32),
        ],
    )
    return k(u, dinv, src_blk, dst_blk)


def _node_body(t_hbm, r_hbm, f0_hbm, dinv_hbm, u_out,
               t0_v, t1_v, f0_v, r0_v, r1_v, di_v, uo_v):
    c = lax.axis_index("c")
    s = lax.axis_index("s")
    w = c * 16 + s
    rows = NP // NW
    base = w * rows
    pltpu.sync_copy(t_hbm.at[0, pl.ds(base, rows)], t0_v)
    pltpu.sync_copy(t_hbm.at[1, pl.ds(base, rows)], t1_v)
    pltpu.sync_copy(f0_hbm.at[pl.ds(base, rows)], f0_v)
    pltpu.sync_copy(r_hbm.at[pl.ds(base, rows)], r0_v)
    pltpu.sync_copy(r_hbm.at[pl.ds(NP + base, rows)], r1_v)
    pltpu.sync_copy(dinv_hbm.at[pl.ds(base, rows)], di_v)

    def grp(g, carry):
        di = di_v[pl.ds(g * 16, 16)]
        rsum = r0_v[pl.ds(g * 16, 16)] + r1_v[pl.ds(g * 16, 16)]
        alpha = 1.0 / (di * rsum + MU)
        av = alpha * di * di
        bv = MU * alpha * di
        for el in range(16):
            i = g * 16 + el
            a = av[el]
            b = bv[el]
            for f in range(HID // 16):
                ts = t0_v[i, pl.ds(f * 16, 16)] + t1_v[i, pl.ds(f * 16, 16)]
                uo_v[i, pl.ds(f * 16, 16)] = (
                    a * ts + b * f0_v[i, pl.ds(f * 16, 16)])
        return carry

    lax.fori_loop(0, rows // 16, grp, 0)
    pltpu.sync_copy(uo_v, u_out.at[pl.ds(base, rows)])


def _node(t_pair, r_pair, f0, dinv):
    rows = NP // NW
    k = pl.kernel(
        _node_body,
        out_type=jax.ShapeDtypeStruct((NP, HID), jnp.float32),
        mesh=_MESH,
        compiler_params=pltpu.CompilerParams(needs_layout_passes=False, use_tc_tiling_on_sc=False),
        scratch_types=[
            pltpu.VMEM((rows, HID), jnp.float32),
            pltpu.VMEM((rows, HID), jnp.float32),
            pltpu.VMEM((rows, HID), jnp.float32),
            pltpu.VMEM((rows,), jnp.float32),
            pltpu.VMEM((rows,), jnp.float32),
            pltpu.VMEM((rows,), jnp.float32),
            pltpu.VMEM((rows, HID), jnp.float32),
        ],
    )
    return k(t_pair, r_pair, f0, dinv)


@jax.jit
def kernel(x, edge_index, W1, b1, W2, b2):
    src = edge_index[0]
    dst = edge_index[1]
    npad = NW * EPT - E
    pad_idx = N + (jnp.arange(npad, dtype=jnp.int32) % (NP - N))
    src_blk = jnp.concatenate([src, pad_idx]).reshape(NW, KC, C)
    dst_blk = jnp.concatenate([dst, pad_idx]).reshape(NW, KC, C)
    x_pad = jnp.pad(x, ((0, NP - N), (0, 0)))

    h = _lin1(x_pad, W1, b1)
    deg_pair = _deg(dst_blk)
    dinv, u = _prep(deg_pair, h)
    for _ in range(K_ITERS):
        t_pair, r_pair = _edge(u, dinv, src_blk, dst_blk)
        u = _node(t_pair, r_pair, h, dinv)
    return _lin2(u[:N], dinv[:N].reshape(N, 1), W2, b2)


# speedup vs baseline: 1.4777x; 1.4777x over previous
import functools

import jax
import jax.numpy as jnp
from jax import lax
from jax.experimental import pallas as pl
from jax.experimental.pallas import tpu as pltpu
from jax.experimental.pallas import tpu_sc as plsc

N = 10000
E = 320000
F_IN = 128
HID = 64
MU = 0.1
EPS = 1e-6
K_ITERS = 4

NW = 32
NP = 10240
EPT = NP
KC = 80
C = 128
STRIPE = NP // 16

_MESH = plsc.VectorSubcoreMesh(core_axis_name="c", subcore_axis_name="s")


def _nrsqrt(v):
    i = lax.bitcast_convert_type(v, jnp.int32)
    i = jnp.int32(0x5F3759DF) - lax.shift_right_logical(i, 1)
    y = lax.bitcast_convert_type(i, jnp.float32)
    for _ in range(2):
        y = y * (1.5 - 0.5 * v * y * y)
    return y


def _lin1_body(x_ref, w_ref, b_ref, o_ref):
    o_ref[...] = jax.nn.relu(
        jnp.dot(x_ref[...], w_ref[...], preferred_element_type=jnp.float32)
        + b_ref[...])


def _lin1(x_pad, W1, b1):
    blk = NP // 8
    return pl.pallas_call(
        _lin1_body,
        grid=(8,),
        in_specs=[
            pl.BlockSpec((blk, F_IN), lambda i: (i, 0)),
            pl.BlockSpec((F_IN, HID), lambda i: (0, 0)),
            pl.BlockSpec((1, HID), lambda i: (0, 0)),
        ],
        out_specs=pl.BlockSpec((blk, HID), lambda i: (i, 0)),
        out_shape=jax.ShapeDtypeStruct((NP, HID), jnp.float32),
    )(x_pad, W1, b1.reshape(1, HID))


def _out_body(u_ref, di_ref, w_ref, b_ref, o_ref):
    f = u_ref[...].astype(jnp.float32) / di_ref[...]
    y = jnp.dot(f, w_ref[...], preferred_element_type=jnp.float32) + b_ref[...]
    m = jnp.max(y, axis=1, keepdims=True)
    ex = jnp.exp(y - m)
    lse = jnp.log(jnp.sum(ex, axis=1, keepdims=True))
    o_ref[...] = y - m - lse


def _lin2(u, dinv_col, W2, b2):
    blk = 1000
    return pl.pallas_call(
        _out_body,
        grid=(10,),
        in_specs=[
            pl.BlockSpec((blk, HID), lambda i: (i, 0), ),
            pl.BlockSpec((blk, 1), lambda i: (i, 0)),
            pl.BlockSpec((HID, HID), lambda i: (0, 0)),
            pl.BlockSpec((1, HID), lambda i: (0, 0)),
        ],
        out_specs=pl.BlockSpec((blk, HID), lambda i: (i, 0)),
        out_shape=jax.ShapeDtypeStruct((N, HID), jnp.float32),
    )(u, dinv_col, W2, b2.reshape(1, HID))


def _deg_body(dst_hbm, deg_out, dst_v, ones_v, zbuf, sem, deg_sp):
    c = lax.axis_index("c")
    s = lax.axis_index("s")
    w = c * 16 + s
    pltpu.sync_copy(dst_hbm.at[w], dst_v)
    for f in range(8):
        ones_v[pl.ds(f * 16, 16)] = jnp.ones((16,), jnp.float32)
    for f in range(STRIPE // 16):
        zbuf[pl.ds(f * 16, 16)] = jnp.zeros((16,), jnp.float32)
    stripe = s * STRIPE
    pltpu.sync_copy(zbuf, deg_sp.at[pl.ds(stripe, STRIPE)])
    plsc.subcore_barrier()

    def chunk(k8, carry):
        descs = [
            pltpu.async_copy(
                ones_v, deg_sp.at[dst_v.at[k8 * 8 + j]], sem, add=True)
            for j in range(8)
        ]
        for d in descs:
            d.wait()
        return carry

    lax.fori_loop(0, KC // 8, chunk, 0)
    plsc.subcore_barrier()
    pltpu.sync_copy(deg_sp.at[pl.ds(stripe, STRIPE)], zbuf)
    pltpu.sync_copy(zbuf, deg_out.at[pl.ds(c * NP + stripe, STRIPE)])


def _deg(dst_blk):
    k = pl.kernel(
        _deg_body,
        out_type=jax.ShapeDtypeStruct((2 * NP,), jnp.float32),
        mesh=_MESH,
        compiler_params=pltpu.CompilerParams(needs_layout_passes=False, use_tc_tiling_on_sc=False),
        scratch_types=[
            pltpu.VMEM((KC, C), jnp.int32),
            pltpu.VMEM((C,), jnp.float32),
            pltpu.VMEM((STRIPE,), jnp.float32),
            pltpu.SemaphoreType.DMA,
            pltpu.VMEM_SHARED((NP,), jnp.float32),
        ],
    )
    return k(dst_blk)


def _prep_body(deg_hbm, h_hbm, dinv_out, u0_out, d0_v, d1_v, h_v, di_v, u0_v):
    c = lax.axis_index("c")
    s = lax.axis_index("s")
    w = c * 16 + s
    base = w * (NP // NW)
    rows = NP // NW
    pltpu.sync_copy(deg_hbm.at[pl.ds(base, rows)], d0_v)
    pltpu.sync_copy(deg_hbm.at[pl.ds(NP + base, rows)], d1_v)
    pltpu.sync_copy(h_hbm.at[pl.ds(base, rows)], h_v)

    def grp(g, carry):
        d = d0_v[pl.ds(g * 16, 16)] + d1_v[pl.ds(g * 16, 16)]
        d = jnp.maximum(d, 1.0)
        dv = _nrsqrt(d)
        di_v[pl.ds(g * 16, 16)] = dv
        for el in range(16):
            i = g * 16 + el
            de = dv[el]
            for f in range(HID // 16):
                u0_v[i, pl.ds(f * 16, 16)] = de * h_v[i, pl.ds(f * 16, 16)]
        return carry

    lax.fori_loop(0, rows // 16, grp, 0)
    pltpu.sync_copy(di_v, dinv_out.at[pl.ds(base, rows)])
    pltpu.sync_copy(u0_v, u0_out.at[pl.ds(base, rows)])


def _prep(deg_pair, h):
    rows = NP // NW
    k = pl.kernel(
        _prep_body,
        out_type=(jax.ShapeDtypeStruct((NP,), jnp.float32),
                  jax.ShapeDtypeStruct((NP, HID), jnp.float32)),
        mesh=_MESH,
        compiler_params=pltpu.CompilerParams(needs_layout_passes=False, use_tc_tiling_on_sc=False),
        scratch_types=[
            pltpu.VMEM((rows,), jnp.float32),
            pltpu.VMEM((rows,), jnp.float32),
            pltpu.VMEM((rows, HID), jnp.float32),
            pltpu.VMEM((rows,), jnp.float32),
            pltpu.VMEM((rows, HID), jnp.float32),
        ],
    )
    return k(deg_pair, h)


def _edge_body(u_hbm, dinv_hbm, src_hbm, dst_hbm, t_out, r_out,
               src_v, dst_v, dinv_vm, us_v0, us_v1, ud_v0, ud_v1,
               msg_v0, msg_v1, pbuf, rbuf0, rbuf1, zbuf,
               gs0, gs1, ss0, ss1, t_sp, r_sp):
    us_v = (us_v0, us_v1)
    ud_v = (ud_v0, ud_v1)
    msg_v = (msg_v0, msg_v1)
    rbuf = (rbuf0, rbuf1)
    gs = (gs0, gs1)
    ss = (ss0, ss1)
    c = lax.axis_index("c")
    s = lax.axis_index("s")
    w = c * 16 + s
    pltpu.sync_copy(dinv_hbm, dinv_vm)
    pltpu.sync_copy(src_hbm.at[w], src_v)
    pltpu.sync_copy(dst_hbm.at[w], dst_v)

    def zmsg(e, carry):
        for f in range(HID // 16):
            msg_v0[e, pl.ds(f * 16, 16)] = jnp.zeros((16,), jnp.float32)
        return carry

    lax.fori_loop(0, C, zmsg, 0)
    for f in range(STRIPE // 16):
        zbuf[pl.ds(f * 16, 16)] = jnp.zeros((16,), jnp.float32)
    stripe = s * STRIPE
    for j in range(STRIPE // C):
        pltpu.sync_copy(msg_v0, t_sp.at[pl.ds(stripe + j * C, C)])
    pltpu.sync_copy(zbuf, r_sp.at[pl.ds(stripe, STRIPE)])
    plsc.subcore_barrier()

    iot = jnp.arange(16, dtype=jnp.int32)

    def compute_chunk(k, usb, udb, msgb, rbb):
        @plsc.parallel_loop(0, C // 16, unroll=2)
        def grp(g):
            eb = g * 16
            pb = g * 256
            for el in range(16):
                e = eb + el
                acc = jnp.zeros((16,), jnp.float32)
                for f in range(HID // 16):
                    d = usb[e, pl.ds(f * 16, 16)] - udb[e, pl.ds(f * 16, 16)]
                    acc = acc + d * d
                pbuf[pl.ds(pb + el * 16, 16)] = acc
            iot16 = iot * 16 + pb
            cols = [plsc.load_gather(pbuf, [iot16 + j]) for j in range(16)]
            while len(cols) > 1:
                cols = [a + b for a, b in zip(cols[::2], cols[1::2])]
            gc = jnp.maximum(cols[0], EPS)
            m = _nrsqrt(gc * _nrsqrt(gc))
            sv = src_v[k, pl.ds(eb, 16)]
            dsv = plsc.load_gather(dinv_vm, [sv])
            rbb[pl.ds(eb, 16)] = m * dsv
            for el in range(16):
                e = eb + el
                me = m[el]
                for f in range(HID // 16):
                    msgb[e, pl.ds(f * 16, 16)] = me * usb[e, pl.ds(f * 16, 16)]

    pltpu.async_copy(u_hbm.at[src_v.at[0]], us_v[0], gs[0])
    pltpu.async_copy(u_hbm.at[dst_v.at[0]], ud_v[0], gs[0])

    def chunk2(k2, carry):
        for b in range(2):
            k = k2 * 2 + b
            pltpu.make_async_copy(u_hbm.at[src_v.at[k]], us_v[b], gs[b]).wait()
            pltpu.make_async_copy(u_hbm.at[dst_v.at[k]], ud_v[b], gs[b]).wait()

            @pl.when(k + 1 < KC)
            def _fire_gather():
                pltpu.async_copy(u_hbm.at[src_v.at[k + 1]], us_v[1 - b], gs[1 - b])
                pltpu.async_copy(u_hbm.at[dst_v.at[k + 1]], ud_v[1 - b], gs[1 - b])

            @pl.when(k2 > 0)
            def _drain_scatter():
                pltpu.make_async_copy(
                    msg_v[b], t_sp.at[dst_v.at[k - 2]], ss[b]).wait()
                pltpu.make_async_copy(
                    rbuf[b], r_sp.at[dst_v.at[k - 2]], ss[b]).wait()

            compute_chunk(k, us_v[b], ud_v[b], msg_v[b], rbuf[b])
            pltpu.async_copy(msg_v[b], t_sp.at[dst_v.at[k]], ss[b], add=True)
            pltpu.async_copy(rbuf[b], r_sp.at[dst_v.at[k]], ss[b], add=True)
        return carry

    lax.fori_loop(0, KC // 2, chunk2, 0)
    for b in range(2):
        pltpu.make_async_copy(
            msg_v[b], t_sp.at[dst_v.at[KC - 2 + b]], ss[b]).wait()
        pltpu.make_async_copy(
            rbuf[b], r_sp.at[dst_v.at[KC - 2 + b]], ss[b]).wait()
    plsc.subcore_barrier()
    pltpu.sync_copy(t_sp.at[pl.ds(stripe, STRIPE)],
                    t_out.at[c, pl.ds(stripe, STRIPE)])
    pltpu.sync_copy(r_sp.at[pl.ds(stripe, STRIPE)],
                    r_out.at[pl.ds(c * NP + stripe, STRIPE)])


def _edge(u, dinv, src_blk, dst_blk):
    k = pl.kernel(
        _edge_body,
        out_type=(jax.ShapeDtypeStruct((2, NP, HID), jnp.float32),
                  jax.ShapeDtypeStruct((2 * NP,), jnp.float32)),
        mesh=_MESH,
        compiler_params=pltpu.CompilerParams(needs_layout_passes=False, use_tc_tiling_on_sc=False),
        scratch_types=[
            pltpu.VMEM((KC, C), jnp.int32),
            pltpu.VMEM((KC, C), jnp.int32),
            pltpu.VMEM((NP,), jnp.float32),
            pltpu.VMEM((C, HID), jnp.float32),
            pltpu.VMEM((C, HID), jnp.float32),
            pltpu.VMEM((C, HID), jnp.float32),
            pltpu.VMEM((C, HID), jnp.float32),
            pltpu.VMEM((C, HID), jnp.float32),
            pltpu.VMEM((C, HID), jnp.float32),
            pltpu.VMEM((2048,), jnp.float32),
            pltpu.VMEM((C,), jnp.float32),
            pltpu.VMEM((C,), jnp.float32),
            pltpu.VMEM((STRIPE,), jnp.float32),
            pltpu.SemaphoreType.DMA,
            pltpu.SemaphoreType.DMA,
            pltpu.SemaphoreType.DMA,
            pltpu.SemaphoreType.DMA,
            pltpu.VMEM_SHARED((NP, HID), jnp.float32),
            pltpu.VMEM_SHARED((NP,), jnp.float32),
        ],
    )
    return k(u, dinv, src_blk, dst_blk)


def _node_body(t_hbm, r_hbm, f0_hbm, dinv_hbm, u_out,
               t0_v, t1_v, f0_v, r0_v, r1_v, di_v, uo_v):
    c = lax.axis_index("c")
    s = lax.axis_index("s")
    w = c * 16 + s
    rows = NP // NW
    base = w * rows
    pltpu.sync_copy(t_hbm.at[0, pl.ds(base, rows)], t0_v)
    pltpu.sync_copy(t_hbm.at[1, pl.ds(base, rows)], t1_v)
    pltpu.sync_copy(f0_hbm.at[pl.ds(base, rows)], f0_v)
    pltpu.sync_copy(r_hbm.at[pl.ds(base, rows)], r0_v)
    pltpu.sync_copy(r_hbm.at[pl.ds(NP + base, rows)], r1_v)
    pltpu.sync_copy(dinv_hbm.at[pl.ds(base, rows)], di_v)

    def grp(g, carry):
        di = di_v[pl.ds(g * 16, 16)]
        rsum = r0_v[pl.ds(g * 16, 16)] + r1_v[pl.ds(g * 16, 16)]
        alpha = 1.0 / (di * rsum + MU)
        av = alpha * di * di
        bv = MU * alpha * di
        for el in range(16):
            i = g * 16 + el
            a = av[el]
            b = bv[el]
            for f in range(HID // 16):
                ts = t0_v[i, pl.ds(f * 16, 16)] + t1_v[i, pl.ds(f * 16, 16)]
                uo_v[i, pl.ds(f * 16, 16)] = (
                    a * ts + b * f0_v[i, pl.ds(f * 16, 16)])
        return carry

    lax.fori_loop(0, rows // 16, grp, 0)
    pltpu.sync_copy(uo_v, u_out.at[pl.ds(base, rows)])


def _node(t_pair, r_pair, f0, dinv):
    rows = NP // NW
    k = pl.kernel(
        _node_body,
        out_type=jax.ShapeDtypeStruct((NP, HID), jnp.float32),
        mesh=_MESH,
        compiler_params=pltpu.CompilerParams(needs_layout_passes=False, use_tc_tiling_on_sc=False),
        scratch_types=[
            pltpu.VMEM((rows, HID), jnp.float32),
            pltpu.VMEM((rows, HID), jnp.float32),
            pltpu.VMEM((rows, HID), jnp.float32),
            pltpu.VMEM((rows,), jnp.float32),
            pltpu.VMEM((rows,), jnp.float32),
            pltpu.VMEM((rows,), jnp.float32),
            pltpu.VMEM((rows, HID), jnp.float32),
        ],
    )
    return k(t_pair, r_pair, f0, dinv)


@jax.jit
def kernel(x, edge_index, W1, b1, W2, b2):
    src = edge_index[0]
    dst = edge_index[1]
    npad = NW * EPT - E
    pad_idx = N + (jnp.arange(npad, dtype=jnp.int32) % (NP - N))
    src_blk = jnp.concatenate([src, pad_idx]).reshape(NW, KC, C)
    dst_blk = jnp.concatenate([dst, pad_idx]).reshape(NW, KC, C)
    x_pad = jnp.pad(x, ((0, NP - N), (0, 0)))

    h = _lin1(x_pad, W1, b1)
    deg_pair = _deg(dst_blk)
    dinv, u = _prep(deg_pair, h)
    for _ in range(K_ITERS):
        t_pair, r_pair = _edge(u, dinv, src_blk, dst_blk)
        u = _node(t_pair, r_pair, h, dinv)
    return _lin2(u[:N], dinv[:N].reshape(N, 1), W2, b2)


# speedup vs baseline: 1.4835x; 1.0039x over previous
import functools

import jax
import jax.numpy as jnp
from jax import lax
from jax.experimental import pallas as pl
from jax.experimental.pallas import tpu as pltpu
from jax.experimental.pallas import tpu_sc as plsc

N = 10000
E = 320000
F_IN = 128
HID = 64
MU = 0.1
EPS = 1e-6
K_ITERS = 4

NW = 32
NP = 10240
EPT = NP
KC = 80
C = 128
STRIPE = NP // 16

_MESH = plsc.VectorSubcoreMesh(core_axis_name="c", subcore_axis_name="s")


def _nrsqrt(v):
    i = lax.bitcast_convert_type(v, jnp.int32)
    i = jnp.int32(0x5F3759DF) - lax.shift_right_logical(i, 1)
    y = lax.bitcast_convert_type(i, jnp.float32)
    for _ in range(2):
        y = y * (1.5 - 0.5 * v * y * y)
    return y


def _lin1_body(x_ref, w_ref, b_ref, o_ref):
    o_ref[...] = jax.nn.relu(
        jnp.dot(x_ref[...], w_ref[...], preferred_element_type=jnp.float32)
        + b_ref[...])


def _lin1(x_pad, W1, b1):
    blk = NP // 8
    return pl.pallas_call(
        _lin1_body,
        grid=(8,),
        in_specs=[
            pl.BlockSpec((blk, F_IN), lambda i: (i, 0)),
            pl.BlockSpec((F_IN, HID), lambda i: (0, 0)),
            pl.BlockSpec((1, HID), lambda i: (0, 0)),
        ],
        out_specs=pl.BlockSpec((blk, HID), lambda i: (i, 0)),
        out_shape=jax.ShapeDtypeStruct((NP, HID), jnp.float32),
    )(x_pad, W1, b1.reshape(1, HID))


def _out_body(u_ref, di_ref, w_ref, b_ref, o_ref):
    f = u_ref[...].astype(jnp.float32) / di_ref[...]
    y = jnp.dot(f, w_ref[...], preferred_element_type=jnp.float32) + b_ref[...]
    m = jnp.max(y, axis=1, keepdims=True)
    ex = jnp.exp(y - m)
    lse = jnp.log(jnp.sum(ex, axis=1, keepdims=True))
    o_ref[...] = y - m - lse


def _lin2(u, dinv_col, W2, b2):
    blk = 1000
    return pl.pallas_call(
        _out_body,
        grid=(10,),
        in_specs=[
            pl.BlockSpec((blk, HID), lambda i: (i, 0), ),
            pl.BlockSpec((blk, 1), lambda i: (i, 0)),
            pl.BlockSpec((HID, HID), lambda i: (0, 0)),
            pl.BlockSpec((1, HID), lambda i: (0, 0)),
        ],
        out_specs=pl.BlockSpec((blk, HID), lambda i: (i, 0)),
        out_shape=jax.ShapeDtypeStruct((N, HID), jnp.float32),
    )(u, dinv_col, W2, b2.reshape(1, HID))


def _deg_body(dst_hbm, deg_out, dst_v, ones_v, zbuf, sem, deg_sp):
    c = lax.axis_index("c")
    s = lax.axis_index("s")
    w = c * 16 + s
    pltpu.sync_copy(dst_hbm.at[w], dst_v)
    for f in range(8):
        ones_v[pl.ds(f * 16, 16)] = jnp.ones((16,), jnp.float32)
    for f in range(STRIPE // 16):
        zbuf[pl.ds(f * 16, 16)] = jnp.zeros((16,), jnp.float32)
    stripe = s * STRIPE
    pltpu.sync_copy(zbuf, deg_sp.at[pl.ds(stripe, STRIPE)])
    plsc.subcore_barrier()

    def chunk(k8, carry):
        descs = [
            pltpu.async_copy(
                ones_v, deg_sp.at[dst_v.at[k8 * 8 + j]], sem, add=True)
            for j in range(8)
        ]
        for d in descs:
            d.wait()
        return carry

    lax.fori_loop(0, KC // 8, chunk, 0)
    plsc.subcore_barrier()
    pltpu.sync_copy(deg_sp.at[pl.ds(stripe, STRIPE)], zbuf)
    pltpu.sync_copy(zbuf, deg_out.at[pl.ds(c * NP + stripe, STRIPE)])


def _deg(dst_blk):
    k = pl.kernel(
        _deg_body,
        out_type=jax.ShapeDtypeStruct((2 * NP,), jnp.float32),
        mesh=_MESH,
        compiler_params=pltpu.CompilerParams(needs_layout_passes=False, use_tc_tiling_on_sc=False),
        scratch_types=[
            pltpu.VMEM((KC, C), jnp.int32),
            pltpu.VMEM((C,), jnp.float32),
            pltpu.VMEM((STRIPE,), jnp.float32),
            pltpu.SemaphoreType.DMA,
            pltpu.VMEM_SHARED((NP,), jnp.float32),
        ],
    )
    return k(dst_blk)


def _prep_body(deg_hbm, h_hbm, dinv_out, u0_out, d0_v, d1_v, h_v, di_v, u0_v):
    c = lax.axis_index("c")
    s = lax.axis_index("s")
    w = c * 16 + s
    base = w * (NP // NW)
    rows = NP // NW
    pltpu.sync_copy(deg_hbm.at[pl.ds(base, rows)], d0_v)
    pltpu.sync_copy(deg_hbm.at[pl.ds(NP + base, rows)], d1_v)
    pltpu.sync_copy(h_hbm.at[pl.ds(base, rows)], h_v)

    def grp(g, carry):
        d = d0_v[pl.ds(g * 16, 16)] + d1_v[pl.ds(g * 16, 16)]
        d = jnp.maximum(d, 1.0)
        dv = _nrsqrt(d)
        di_v[pl.ds(g * 16, 16)] = dv
        for el in range(16):
            i = g * 16 + el
            de = dv[el]
            for f in range(HID // 16):
                u0_v[i, pl.ds(f * 16, 16)] = de * h_v[i, pl.ds(f * 16, 16)]
        return carry

    lax.fori_loop(0, rows // 16, grp, 0)
    pltpu.sync_copy(di_v, dinv_out.at[pl.ds(base, rows)])
    pltpu.sync_copy(u0_v, u0_out.at[pl.ds(base, rows)])


def _prep(deg_pair, h):
    rows = NP // NW
    k = pl.kernel(
        _prep_body,
        out_type=(jax.ShapeDtypeStruct((NP,), jnp.float32),
                  jax.ShapeDtypeStruct((NP, HID), jnp.float32)),
        mesh=_MESH,
        compiler_params=pltpu.CompilerParams(needs_layout_passes=False, use_tc_tiling_on_sc=False),
        scratch_types=[
            pltpu.VMEM((rows,), jnp.float32),
            pltpu.VMEM((rows,), jnp.float32),
            pltpu.VMEM((rows, HID), jnp.float32),
            pltpu.VMEM((rows,), jnp.float32),
            pltpu.VMEM((rows, HID), jnp.float32),
        ],
    )
    return k(deg_pair, h)


def _edge_body(u_hbm, dinv_hbm, src_hbm, dst_hbm, t_out, r_out,
               src_v, dst_v, dinv_vm, us_v0, us_v1, ud_v0, ud_v1,
               msg_v0, msg_v1, pbuf, rbuf0, rbuf1, zbuf,
               gs0, gs1, ss0, ss1, t_sp, r_sp):
    us_v = (us_v0, us_v1)
    ud_v = (ud_v0, ud_v1)
    msg_v = (msg_v0, msg_v1)
    rbuf = (rbuf0, rbuf1)
    gs = (gs0, gs1)
    ss = (ss0, ss1)
    c = lax.axis_index("c")
    s = lax.axis_index("s")
    w = c * 16 + s
    pltpu.sync_copy(dinv_hbm, dinv_vm)
    pltpu.sync_copy(src_hbm.at[w], src_v)
    pltpu.sync_copy(dst_hbm.at[w], dst_v)

    def zmsg(e, carry):
        for f in range(HID // 16):
            msg_v0[e, pl.ds(f * 16, 16)] = jnp.zeros((16,), jnp.float32)
        return carry

    lax.fori_loop(0, C, zmsg, 0)
    for f in range(STRIPE // 16):
        zbuf[pl.ds(f * 16, 16)] = jnp.zeros((16,), jnp.float32)
    stripe = s * STRIPE
    for j in range(STRIPE // C):
        pltpu.sync_copy(msg_v0, t_sp.at[pl.ds(stripe + j * C, C)])
    pltpu.sync_copy(zbuf, r_sp.at[pl.ds(stripe, STRIPE)])
    plsc.subcore_barrier()

    iot = jnp.arange(16, dtype=jnp.int32)

    def compute_chunk(k, usb, udb, msgb, rbb):
        @plsc.parallel_loop(0, C // 16, unroll=2)
        def grp(g):
            eb = g * 16
            pb = g * 256
            for el in range(16):
                e = eb + el
                acc = jnp.zeros((16,), jnp.float32)
                for f in range(HID // 16):
                    d = usb[e, pl.ds(f * 16, 16)] - udb[e, pl.ds(f * 16, 16)]
                    acc = acc + d * d
                pbuf[pl.ds(pb + el * 16, 16)] = acc
            iot16 = iot * 16 + pb
            cols = [plsc.load_gather(pbuf, [iot16 + j]) for j in range(16)]
            while len(cols) > 1:
                cols = [a + b for a, b in zip(cols[::2], cols[1::2])]
            gc = jnp.maximum(cols[0], EPS)
            m = _nrsqrt(gc * _nrsqrt(gc))
            sv = src_v[k, pl.ds(eb, 16)]
            dsv = plsc.load_gather(dinv_vm, [sv])
            rbb[pl.ds(eb, 16)] = m * dsv
            for el in range(16):
                e = eb + el
                me = m[el]
                for f in range(HID // 16):
                    msgb[e, pl.ds(f * 16, 16)] = me * usb[e, pl.ds(f * 16, 16)]

    pltpu.async_copy(u_hbm.at[src_v.at[0]], us_v[0], gs[0])
    pltpu.async_copy(u_hbm.at[dst_v.at[0]], ud_v[0], gs[0])

    def chunk2(k2, carry):
        for b in range(2):
            k = k2 * 2 + b
            pltpu.make_async_copy(u_hbm.at[src_v.at[k]], us_v[b], gs[b]).wait()
            pltpu.make_async_copy(u_hbm.at[dst_v.at[k]], ud_v[b], gs[b]).wait()

            @pl.when(k + 1 < KC)
            def _fire_gather():
                pltpu.async_copy(u_hbm.at[src_v.at[k + 1]], us_v[1 - b], gs[1 - b])
                pltpu.async_copy(u_hbm.at[dst_v.at[k + 1]], ud_v[1 - b], gs[1 - b])

            @pl.when(k2 > 0)
            def _drain_scatter():
                pltpu.make_async_copy(
                    msg_v[b], t_sp.at[dst_v.at[k - 2]], ss[b]).wait()
                pltpu.make_async_copy(
                    rbuf[b], r_sp.at[dst_v.at[k - 2]], ss[b]).wait()

            compute_chunk(k, us_v[b], ud_v[b], msg_v[b], rbuf[b])
            pltpu.async_copy(msg_v[b], t_sp.at[dst_v.at[k]], ss[b], add=True)
            pltpu.async_copy(rbuf[b], r_sp.at[dst_v.at[k]], ss[b], add=True)
        return carry

    lax.fori_loop(0, KC // 2, chunk2, 0)
    for b in range(2):
        pltpu.make_async_copy(
            msg_v[b], t_sp.at[dst_v.at[KC - 2 + b]], ss[b]).wait()
        pltpu.make_async_copy(
            rbuf[b], r_sp.at[dst_v.at[KC - 2 + b]], ss[b]).wait()
    plsc.subcore_barrier()
    for j in range(STRIPE // C):
        pltpu.sync_copy(t_sp.at[pl.ds(stripe + j * C, C)], msg_v0)
        pltpu.sync_copy(msg_v0, t_out.at[c, pl.ds(stripe + j * C, C)])
    pltpu.sync_copy(r_sp.at[pl.ds(stripe, STRIPE)], zbuf)
    pltpu.sync_copy(zbuf, r_out.at[pl.ds(c * NP + stripe, STRIPE)])


def _edge(u, dinv, src_blk, dst_blk):
    k = pl.kernel(
        _edge_body,
        out_type=(jax.ShapeDtypeStruct((2, NP, HID), jnp.float32),
                  jax.ShapeDtypeStruct((2 * NP,), jnp.float32)),
        mesh=_MESH,
        compiler_params=pltpu.CompilerParams(needs_layout_passes=False, use_tc_tiling_on_sc=False),
        scratch_types=[
            pltpu.VMEM((KC, C), jnp.int32),
            pltpu.VMEM((KC, C), jnp.int32),
            pltpu.VMEM((NP,), jnp.float32),
            pltpu.VMEM((C, HID), jnp.float32),
            pltpu.VMEM((C, HID), jnp.float32),
            pltpu.VMEM((C, HID), jnp.float32),
            pltpu.VMEM((C, HID), jnp.float32),
            pltpu.VMEM((C, HID), jnp.float32),
            pltpu.VMEM((C, HID), jnp.float32),
            pltpu.VMEM((2048,), jnp.float32),
            pltpu.VMEM((C,), jnp.float32),
            pltpu.VMEM((C,), jnp.float32),
            pltpu.VMEM((STRIPE,), jnp.float32),
            pltpu.SemaphoreType.DMA,
            pltpu.SemaphoreType.DMA,
            pltpu.SemaphoreType.DMA,
            pltpu.SemaphoreType.DMA,
            pltpu.VMEM_SHARED((NP, HID), jnp.float32),
            pltpu.VMEM_SHARED((NP,), jnp.float32),
        ],
    )
    return k(u, dinv, src_blk, dst_blk)


def _node_body(t_hbm, r_hbm, f0_hbm, dinv_hbm, u_out,
               t0_v, t1_v, f0_v, r0_v, r1_v, di_v, uo_v):
    c = lax.axis_index("c")
    s = lax.axis_index("s")
    w = c * 16 + s
    rows = NP // NW
    base = w * rows
    pltpu.sync_copy(t_hbm.at[0, pl.ds(base, rows)], t0_v)
    pltpu.sync_copy(t_hbm.at[1, pl.ds(base, rows)], t1_v)
    pltpu.sync_copy(f0_hbm.at[pl.ds(base, rows)], f0_v)
    pltpu.sync_copy(r_hbm.at[pl.ds(base, rows)], r0_v)
    pltpu.sync_copy(r_hbm.at[pl.ds(NP + base, rows)], r1_v)
    pltpu.sync_copy(dinv_hbm.at[pl.ds(base, rows)], di_v)

    def grp(g, carry):
        di = di_v[pl.ds(g * 16, 16)]
        rsum = r0_v[pl.ds(g * 16, 16)] + r1_v[pl.ds(g * 16, 16)]
        alpha = 1.0 / (di * rsum + MU)
        av = alpha * di * di
        bv = MU * alpha * di
        for el in range(16):
            i = g * 16 + el
            a = av[el]
            b = bv[el]
            for f in range(HID // 16):
                ts = t0_v[i, pl.ds(f * 16, 16)] + t1_v[i, pl.ds(f * 16, 16)]
                uo_v[i, pl.ds(f * 16, 16)] = (
                    a * ts + b * f0_v[i, pl.ds(f * 16, 16)])
        return carry

    lax.fori_loop(0, rows // 16, grp, 0)
    pltpu.sync_copy(uo_v, u_out.at[pl.ds(base, rows)])


def _node(t_pair, r_pair, f0, dinv):
    rows = NP // NW
    k = pl.kernel(
        _node_body,
        out_type=jax.ShapeDtypeStruct((NP, HID), jnp.float32),
        mesh=_MESH,
        compiler_params=pltpu.CompilerParams(needs_layout_passes=False, use_tc_tiling_on_sc=False),
        scratch_types=[
            pltpu.VMEM((rows, HID), jnp.float32),
            pltpu.VMEM((rows, HID), jnp.float32),
            pltpu.VMEM((rows, HID), jnp.float32),
            pltpu.VMEM((rows,), jnp.float32),
            pltpu.VMEM((rows,), jnp.float32),
            pltpu.VMEM((rows,), jnp.float32),
            pltpu.VMEM((rows, HID), jnp.float32),
        ],
    )
    return k(t_pair, r_pair, f0, dinv)


@jax.jit
def kernel(x, edge_index, W1, b1, W2, b2):
    src = edge_index[0]
    dst = edge_index[1]
    npad = NW * EPT - E
    pad_idx = N + (jnp.arange(npad, dtype=jnp.int32) % (NP - N))
    src_blk = jnp.concatenate([src, pad_idx]).reshape(NW, KC, C)
    dst_blk = jnp.concatenate([dst, pad_idx]).reshape(NW, KC, C)
    x_pad = jnp.pad(x, ((0, NP - N), (0, 0)))

    h = _lin1(x_pad, W1, b1)
    deg_pair = _deg(dst_blk)
    dinv, u = _prep(deg_pair, h)
    for _ in range(K_ITERS):
        t_pair, r_pair = _edge(u, dinv, src_blk, dst_blk)
        u = _node(t_pair, r_pair, h, dinv)
    return _lin2(u[:N], dinv[:N].reshape(N, 1), W2, b2)
